# baseline (speedup 1.0000x reference)
import functools

import jax
import jax.numpy as jnp
from jax import lax
from jax.experimental import pallas as pl
from jax.experimental.pallas import tpu as pltpu
from jax.experimental.pallas import tpu_sc as plsc

EMB_DIM = 32
CHUNK = 512
GROUP = 10


@functools.lru_cache(maxsize=None)
def _make_table_transpose(V, D):
    info = plsc.get_sparse_core_info()
    NC, NS = info.num_cores, info.num_subcores
    NW = NC * NS
    W = 128
    n_full = V // W
    rem = V - n_full * W
    mesh = plsc.VectorSubcoreMesh(core_axis_name="c", subcore_axis_name="s")

    @functools.partial(
        pl.kernel,
        mesh=mesh,
        compiler_params=pltpu.CompilerParams(needs_layout_passes=False),
        out_type=jax.ShapeDtypeStruct((V * D,), jnp.float32),
        scratch_types=[
            pltpu.VMEM((D // 8, 8, W), jnp.float32),
            pltpu.VMEM((W * (D + 1),), jnp.float32),
            pltpu.VMEM((W * D,), jnp.float32),
            pltpu.SemaphoreType.DMA,
        ],
    )
    def transpose_kernel(src_hbm, tail_hbm, out_hbm, blk_v, tr_v, pk_v, sem):
        wid = lax.axis_index("s") * NC + lax.axis_index("c")
        iota = lax.iota(jnp.int32, 16)
        iota_p = iota * (D + 1)
        my_n = (n_full - wid + NW - 1) // NW

        @pl.when(wid == 0)
        def _copy_tail():
            pltpu.sync_copy(tail_hbm, out_hbm.at[pl.ds(n_full * W * D, rem * D)])

        def body(i, carry):
            j = wid + i * NW
            c0 = pl.multiple_of(j * W, W)
            for tb in range(D // 8):
                pltpu.async_copy(
                    src_hbm.at[pl.ds(tb * 8, 8), pl.ds(c0, W)], blk_v.at[tb], sem
                ).wait()

            @plsc.parallel_loop(0, D, unroll=4, carry=None)
            def scat(d):
                for cg in range(W // 16):
                    v = blk_v[d // 8, d % 8, pl.ds(cg * 16, 16)]
                    plsc.store_scatter(
                        tr_v, [iota_p + (cg * 16 * (D + 1) + d)], v
                    )

            @plsc.parallel_loop(0, W, unroll=4, carry=None)
            def compact(c):
                for h in range(D // 16):
                    pk_v[pl.ds(c * D + h * 16, 16)] = tr_v[
                        pl.ds(c * (D + 1) + h * 16, 16)
                    ]

            pltpu.sync_copy(pk_v, out_hbm.at[pl.ds(c0 * D, W * D)])
            return carry

        lax.fori_loop(0, my_n, body, 0)

    return transpose_kernel


@functools.lru_cache(maxsize=None)
def _make_gather(T, B, D):
    info = plsc.get_sparse_core_info()
    NC, NS = info.num_cores, info.num_subcores
    NW = NC * NS
    C = CHUNK
    blocks_per_t = B // C
    n_blocks = T * blocks_per_t
    assert n_blocks % (NW * GROUP) == 0
    per_w = n_blocks // NW
    n_groups = per_w // GROUP
    mesh = plsc.VectorSubcoreMesh(core_axis_name="c", subcore_axis_name="s")

    @functools.partial(
        pl.kernel,
        mesh=mesh,
        compiler_params=pltpu.CompilerParams(
            use_tc_tiling_on_sc=False, needs_layout_passes=False
        ),
        out_type=jax.ShapeDtypeStruct((T, D, B), jnp.float32),
        scratch_types=[
            pltpu.VMEM((GROUP * C,), jnp.int32),
            pltpu.VMEM((2, C, D), jnp.float32),
            pltpu.VMEM((2, D, C + 1), jnp.float32),
            pltpu.SemaphoreType.DMA,
            pltpu.SemaphoreType.DMA,
            pltpu.SemaphoreType.DMA,
        ],
    )
    def gather_kernel(
        idx_hbm, table_hbm, out_hbm, idxg_v, rows_v, tr_v, sem_g0, sem_g1, sem_o
    ):
        wid = lax.axis_index("s") * NC + lax.axis_index("c")
        base = wid * per_w
        iota = lax.iota(jnp.int32, 16)
        sem_g = (sem_g0, sem_g1)

        iota_hi = iota + 16

        def tpose(b):
            @plsc.parallel_loop(0, C, unroll=8, carry=None)
            def tbody(r):
                rcol = jnp.full((16,), 0, jnp.int32) + r
                v0 = rows_v[b, r, pl.ds(0, 16)]
                v1 = rows_v[b, r, pl.ds(16, 16)]
                plsc.store_scatter(tr_v.at[b], [iota, rcol], v0)
                plsc.store_scatter(tr_v.at[b], [iota_hi, rcol], v1)

        def group_body(grp, carry):
            g0 = base + grp * GROUP
            pltpu.sync_copy(idx_hbm.at[pl.ds(g0 * C, GROUP * C)], idxg_v)
            gather = [None] * GROUP
            out_dma = [None] * GROUP
            gather[0] = pltpu.async_copy(
                table_hbm.at[idxg_v.at[pl.ds(0, C)]], rows_v.at[0], sem_g[0]
            )
            for k in range(GROUP):
                b = k % 2
                if k + 1 < GROUP:
                    gather[k + 1] = pltpu.async_copy(
                        table_hbm.at[idxg_v.at[pl.ds((k + 1) * C, C)]],
                        rows_v.at[(k + 1) % 2],
                        sem_g[(k + 1) % 2],
                    )
                gather[k].wait()
                if k >= 2:
                    out_dma[k - 2].wait()
                tpose(b)
                g = g0 + k
                t = g // blocks_per_t
                b0 = (g % blocks_per_t) * C
                out_dma[k] = pltpu.async_copy(
                    tr_v.at[b, :, pl.ds(0, C)], out_hbm.at[t, :, pl.ds(b0, C)], sem_o
                )
            out_dma[GROUP - 2].wait()
            out_dma[GROUP - 1].wait()
            return carry

        lax.fori_loop(0, n_groups, group_body, 0)

    return gather_kernel


def kernel(term_ids, emb_weight):
    B, T = term_ids.shape
    V, D = emb_weight.shape
    idx_flat = term_ids.T.astype(jnp.int32).reshape(-1)
    n_full = V // 128
    tail_flat = emb_weight[n_full * 128:].reshape(-1)
    t_lin = _make_table_transpose(V, D)(emb_weight.T, tail_flat)
    t_lin = jax.lax.optimization_barrier(t_lin)
    out = _make_gather(T, B, EMB_DIM)(idx_flat, t_lin.reshape(V, D))
    return out.transpose(2, 0, 1)


# speedup vs baseline: 1.4879x; 1.4879x over previous
import functools

import jax
import jax.numpy as jnp
from jax import lax
from jax.experimental import pallas as pl
from jax.experimental.pallas import tpu as pltpu
from jax.experimental.pallas import tpu_sc as plsc

EMB_DIM = 32
CHUNK = 512
GROUP = 10


@functools.lru_cache(maxsize=None)
def _make_table_transpose(V, D):
    info = plsc.get_sparse_core_info()
    NC, NS = info.num_cores, info.num_subcores
    NW = NC * NS
    W = 128
    n_full = V // W
    rem = V - n_full * W
    mesh = plsc.VectorSubcoreMesh(core_axis_name="c", subcore_axis_name="s")

    @functools.partial(
        pl.kernel,
        mesh=mesh,
        compiler_params=pltpu.CompilerParams(needs_layout_passes=False),
        out_type=jax.ShapeDtypeStruct((V * D,), jnp.float32),
        scratch_types=[
            pltpu.VMEM((D // 8, 8, W), jnp.float32),
            pltpu.VMEM((W * (D + 1),), jnp.float32),
            pltpu.VMEM((W * D,), jnp.float32),
            pltpu.SemaphoreType.DMA,
        ],
    )
    def transpose_kernel(src_hbm, tail_hbm, out_hbm, blk_v, tr_v, pk_v, sem):
        wid = lax.axis_index("s") * NC + lax.axis_index("c")
        iota = lax.iota(jnp.int32, 16)
        iota_p = iota * (D + 1)
        my_n = (n_full - wid + NW - 1) // NW

        @pl.when(wid == 0)
        def _copy_tail():
            pltpu.sync_copy(tail_hbm, out_hbm.at[pl.ds(n_full * W * D, rem * D)])

        def body(i, carry):
            j = wid + i * NW
            c0 = pl.multiple_of(j * W, W)
            dmas = [
                pltpu.async_copy(
                    src_hbm.at[pl.ds(tb * 8, 8), pl.ds(c0, W)], blk_v.at[tb], sem
                )
                for tb in range(D // 8)
            ]
            for h in dmas:
                h.wait()

            @plsc.parallel_loop(0, D, unroll=4, carry=None)
            def scat(d):
                for cg in range(W // 16):
                    v = blk_v[d // 8, d % 8, pl.ds(cg * 16, 16)]
                    plsc.store_scatter(
                        tr_v, [iota_p + (cg * 16 * (D + 1) + d)], v
                    )

            @plsc.parallel_loop(0, W, unroll=4, carry=None)
            def compact(c):
                for h in range(D // 16):
                    pk_v[pl.ds(c * D + h * 16, 16)] = tr_v[
                        pl.ds(c * (D + 1) + h * 16, 16)
                    ]

            pltpu.sync_copy(pk_v, out_hbm.at[pl.ds(c0 * D, W * D)])
            return carry

        lax.fori_loop(0, my_n, body, 0)

    return transpose_kernel


@functools.lru_cache(maxsize=None)
def _make_gather(T, B, D):
    info = plsc.get_sparse_core_info()
    NC, NS = info.num_cores, info.num_subcores
    NW = NC * NS
    C = CHUNK
    blocks_per_t = B // C
    n_blocks = T * blocks_per_t
    assert n_blocks % (NW * GROUP) == 0
    per_w = n_blocks // NW
    n_groups = per_w // GROUP
    mesh = plsc.VectorSubcoreMesh(core_axis_name="c", subcore_axis_name="s")

    @functools.partial(
        pl.kernel,
        mesh=mesh,
        compiler_params=pltpu.CompilerParams(
            use_tc_tiling_on_sc=False, needs_layout_passes=False
        ),
        out_type=jax.ShapeDtypeStruct((T, D, B), jnp.float32),
        scratch_types=[
            pltpu.VMEM((GROUP * C,), jnp.int32),
            pltpu.VMEM((2, C, D), jnp.float32),
            pltpu.VMEM((2, D, C + 1), jnp.float32),
            pltpu.SemaphoreType.DMA,
            pltpu.SemaphoreType.DMA,
            pltpu.SemaphoreType.DMA,
        ],
    )
    def gather_kernel(
        idx_hbm, table_hbm, out_hbm, idxg_v, rows_v, tr_v, sem_g0, sem_g1, sem_o
    ):
        wid = lax.axis_index("s") * NC + lax.axis_index("c")
        base = wid * per_w
        iota = lax.iota(jnp.int32, 16)
        sem_g = (sem_g0, sem_g1)

        iota_hi = iota + 16

        def tpose(b):
            @plsc.parallel_loop(0, C, unroll=8, carry=None)
            def tbody(r):
                rcol = jnp.full((16,), 0, jnp.int32) + r
                v0 = rows_v[b, r, pl.ds(0, 16)]
                v1 = rows_v[b, r, pl.ds(16, 16)]
                plsc.store_scatter(tr_v.at[b], [iota, rcol], v0)
                plsc.store_scatter(tr_v.at[b], [iota_hi, rcol], v1)

        def group_body(grp, carry):
            g0 = base + grp * GROUP
            pltpu.sync_copy(idx_hbm.at[pl.ds(g0 * C, GROUP * C)], idxg_v)
            gather = [None] * GROUP
            out_dma = [None] * GROUP
            gather[0] = pltpu.async_copy(
                table_hbm.at[idxg_v.at[pl.ds(0, C)]], rows_v.at[0], sem_g[0]
            )
            for k in range(GROUP):
                b = k % 2
                if k + 1 < GROUP:
                    gather[k + 1] = pltpu.async_copy(
                        table_hbm.at[idxg_v.at[pl.ds((k + 1) * C, C)]],
                        rows_v.at[(k + 1) % 2],
                        sem_g[(k + 1) % 2],
                    )
                gather[k].wait()
                if k >= 2:
                    out_dma[k - 2].wait()
                tpose(b)
                g = g0 + k
                t = g // blocks_per_t
                b0 = (g % blocks_per_t) * C
                out_dma[k] = pltpu.async_copy(
                    tr_v.at[b, :, pl.ds(0, C)], out_hbm.at[t, :, pl.ds(b0, C)], sem_o
                )
            out_dma[GROUP - 2].wait()
            out_dma[GROUP - 1].wait()
            return carry

        lax.fori_loop(0, n_groups, group_body, 0)

    return gather_kernel


def kernel(term_ids, emb_weight):
    B, T = term_ids.shape
    V, D = emb_weight.shape
    idx_flat = term_ids.T.astype(jnp.int32).reshape(-1)
    n_full = V // 128
    tail_flat = emb_weight[n_full * 128:].reshape(-1)
    t_lin = _make_table_transpose(V, D)(emb_weight.T, tail_flat)
    t_lin = jax.lax.optimization_barrier(t_lin)
    out = _make_gather(T, B, EMB_DIM)(idx_flat, t_lin.reshape(V, D))
    return out.transpose(2, 0, 1)
